# baseline (speedup 1.0000x reference)
import functools

import jax
import jax.numpy as jnp
from jax import lax
from jax.experimental import pallas as pl
from jax.experimental.pallas import tpu as pltpu
from jax.experimental.pallas import tpu_sc as plsc

ALPHA = 0.2
NC, NS = 2, 16
NW = NC * NS
CH = 384
NBUF = 4


def _leaky_exp(x):
    return jnp.exp(jnp.maximum(x, ALPHA * x))


def _tc_layer0(h, W0, W1, a0, a1, npad, grid=9):
    n, d = h.shape
    nh = W0.shape[1]
    f2 = 2 * nh
    blk = npad // grid

    def body(h_ref, w0_ref, w1_ref, a0_ref, a1_ref, wh_ref, as0_ref,
             ad0_ref, as1_ref, ad1_ref):
        wh0 = h_ref[...] @ w0_ref[...]
        wh1 = h_ref[...] @ w1_ref[...]
        wh_ref[...] = jnp.concatenate([wh0, wh1], axis=1)
        as0_ref[...] = wh0 @ a0_ref[...][:nh, :]
        ad0_ref[...] = wh0 @ a0_ref[...][nh:, :]
        as1_ref[...] = wh1 @ a1_ref[...][:nh, :]
        ad1_ref[...] = wh1 @ a1_ref[...][nh:, :]

    vec = jax.ShapeDtypeStruct((npad, 1), jnp.float32)
    return pl.pallas_call(
        body,
        grid=(grid,),
        in_specs=[
            pl.BlockSpec((blk, d), lambda i: (i, 0)),
            pl.BlockSpec((d, nh), lambda i: (0, 0)),
            pl.BlockSpec((d, nh), lambda i: (0, 0)),
            pl.BlockSpec(a0.shape, lambda i: (0, 0)),
            pl.BlockSpec(a1.shape, lambda i: (0, 0)),
        ],
        out_specs=[
            pl.BlockSpec((blk, f2), lambda i: (i, 0)),
            pl.BlockSpec((blk, 1), lambda i: (i, 0)),
            pl.BlockSpec((blk, 1), lambda i: (i, 0)),
            pl.BlockSpec((blk, 1), lambda i: (i, 0)),
            pl.BlockSpec((blk, 1), lambda i: (i, 0)),
        ],
        out_shape=[jax.ShapeDtypeStruct((npad, f2), jnp.float32),
                   vec, vec, vec, vec],
    )(h, W0, W1, a0, a1)


def _tc_mid(numer, den0, den1, W_out, a_out, grid=9):
    _, npad, f2 = numer.shape
    nh = f2 // 2
    nc = W_out.shape[1]
    blk = npad // grid

    def body(num_ref, d0_ref, d1_ref, w_ref, a_ref, whx_ref, aso_ref,
             ado_ref):
        num = num_ref[0] + num_ref[1]
        d0 = d0_ref[0, :] + d0_ref[1, :]
        d1 = d1_ref[0, :] + d1_ref[1, :]
        d0 = jnp.where(d0 == 0.0, 1.0, d0)
        d1 = jnp.where(d1 == 0.0, 1.0, d1)
        x0 = num[:, :nh] / d0[:, None]
        x1 = num[:, nh:] / d1[:, None]
        x = jnp.concatenate([x0, x1], axis=1)
        x = jnp.where(x > 0.0, x, jnp.exp(x) - 1.0)
        whx = x @ w_ref[...]
        whx_ref[...] = whx
        aso_ref[...] = whx @ a_ref[...][:nc, :]
        ado_ref[...] = whx @ a_ref[...][nc:, :]

    vec = jax.ShapeDtypeStruct((npad, 1), jnp.float32)
    return pl.pallas_call(
        body,
        grid=(grid,),
        in_specs=[
            pl.BlockSpec((2, blk, f2), lambda i: (0, i, 0)),
            pl.BlockSpec((2, blk), lambda i: (0, i)),
            pl.BlockSpec((2, blk), lambda i: (0, i)),
            pl.BlockSpec(W_out.shape, lambda i: (0, 0)),
            pl.BlockSpec(a_out.shape, lambda i: (0, 0)),
        ],
        out_specs=[
            pl.BlockSpec((blk, nc), lambda i: (i, 0)),
            pl.BlockSpec((blk, 1), lambda i: (i, 0)),
            pl.BlockSpec((blk, 1), lambda i: (i, 0)),
        ],
        out_shape=[jax.ShapeDtypeStruct((npad, nc), jnp.float32), vec, vec],
    )(numer, den0, den1, W_out, a_out)


def _tc_final(numer, den, n, grid=9):
    _, npad, nc = numer.shape
    blk = npad // grid

    def body(num_ref, d_ref, out_ref):
        num = num_ref[0] + num_ref[1]
        d = d_ref[0, :] + d_ref[1, :]
        d = jnp.where(d == 0.0, 1.0, d)
        x = num / d[:, None]
        x = x - jnp.max(x, axis=1, keepdims=True)
        out_ref[...] = x - jnp.log(jnp.sum(jnp.exp(x), axis=1, keepdims=True))

    return pl.pallas_call(
        body,
        grid=(grid,),
        in_specs=[
            pl.BlockSpec((2, blk, nc), lambda i: (0, i, 0)),
            pl.BlockSpec((2, blk), lambda i: (0, i)),
        ],
        out_specs=pl.BlockSpec((blk, nc), lambda i: (i, 0)),
        out_shape=jax.ShapeDtypeStruct((n, nc), jnp.float32),
    )(numer, den)


def _sc_edge(two_heads, npad, f2, e):
    nh = f2 // 2
    nchunks = e // CH
    cpt = (nchunks + NW - 1) // NW
    zch = npad // CH
    zpt = (zch + NS - 1) // NS

    mesh = plsc.VectorSubcoreMesh(core_axis_name="c", subcore_axis_name="s",
                                  num_cores=NC)
    vecf = jax.ShapeDtypeStruct((NC * npad,), jnp.float32)
    out_type = [jax.ShapeDtypeStruct((NC, npad, f2), jnp.float32), vecf, vecf]
    scratch = [
        pltpu.VMEM((npad,), jnp.float32),
        pltpu.VMEM((npad,), jnp.float32),
        pltpu.VMEM((npad,), jnp.float32),
        pltpu.VMEM((npad,), jnp.float32),
        pltpu.VMEM((NBUF, CH), jnp.int32),
        pltpu.VMEM((NBUF, CH), jnp.int32),
        pltpu.VMEM((NBUF, CH, f2), jnp.float32),
        pltpu.VMEM((NBUF, CH), jnp.float32),
        pltpu.VMEM((NBUF, CH), jnp.float32),
        pltpu.VMEM_SHARED((npad, f2), jnp.float32),
        pltpu.VMEM_SHARED((npad,), jnp.float32),
        pltpu.VMEM_SHARED((npad,), jnp.float32),
        pltpu.SemaphoreType.DMA,
        pltpu.SemaphoreType.DMA,
        pltpu.SemaphoreType.DMA,
    ]

    def body(wh_h, as0_h, ad0_h, as1_h, ad1_h, src_h, dst_h,
             numer_o, den0_o, den1_o,
             as0_v, ad0_v, as1_v, ad1_v, src_v, dst_v, rows_v, w0_v, w1_v,
             numer_sh, den0_sh, den1_sh, sem_a, sem_b, sem_d):
        c = lax.axis_index("c")
        s = lax.axis_index("s")
        zero16 = jnp.zeros((16,), jnp.float32)

        alpha_cps = [pltpu.make_async_copy(as0_h, as0_v, sem_a),
                     pltpu.make_async_copy(ad0_h, ad0_v, sem_a)]
        if two_heads:
            alpha_cps += [pltpu.make_async_copy(as1_h, as1_v, sem_a),
                          pltpu.make_async_copy(ad1_h, ad1_v, sem_a)]
        for cp in alpha_cps:
            cp.start()

        def zrow(i, _):
            rows_v[0, i, pl.ds(0, 16)] = zero16
            rows_v[0, i, pl.ds(nh, 16)] = zero16
            return _
        lax.fori_loop(0, CH, zrow, None)
        for g in range(CH // 16):
            w0_v[0, pl.ds(g * 16, 16)] = zero16

        def zacc(k, _):
            ch = s + k * NS
            @pl.when(ch < zch)
            def _do():
                sl = pl.ds(ch * CH, CH)
                pltpu.async_copy(rows_v.at[0], numer_sh.at[sl], sem_d)
                pltpu.async_copy(w0_v.at[0], den0_sh.at[sl], sem_d)
                if two_heads:
                    pltpu.async_copy(w0_v.at[0], den1_sh.at[sl], sem_d)
            return _
        lax.fori_loop(0, zpt, zacc, None)

        def zwait(k, _):
            ch = s + k * NS
            @pl.when(ch < zch)
            def _do():
                sl = pl.ds(ch * CH, CH)
                pltpu.make_async_copy(rows_v.at[0], numer_sh.at[sl],
                                      sem_d).wait()
                pltpu.make_async_copy(w0_v.at[0], den0_sh.at[sl],
                                      sem_d).wait()
                if two_heads:
                    pltpu.make_async_copy(w0_v.at[0], den1_sh.at[sl],
                                          sem_d).wait()
            return _
        lax.fori_loop(0, zpt, zwait, None)
        for cp in alpha_cps:
            cp.wait()
        plsc.subcore_barrier()

        wid = c * NS + s

        def _valid(t):
            return wid + t * NW < nchunks

        def _idx_copies(t, b):
            base = (wid + t * NW) * CH
            return (pltpu.make_async_copy(src_h.at[pl.ds(base, CH)],
                                          src_v.at[b], sem_a),
                    pltpu.make_async_copy(dst_h.at[pl.ds(base, CH)],
                                          dst_v.at[b], sem_a))

        def _gather_copy(b):
            return pltpu.make_async_copy(wh_h.at[src_v.at[b]], rows_v.at[b],
                                         sem_b)

        def _scatter_copies(b):
            cps = [pltpu.make_async_copy(rows_v.at[b],
                                         numer_sh.at[dst_v.at[b]], sem_d),
                   pltpu.make_async_copy(w0_v.at[b],
                                         den0_sh.at[dst_v.at[b]], sem_d)]
            if two_heads:
                cps.append(pltpu.make_async_copy(w1_v.at[b],
                                                 den1_sh.at[dst_v.at[b]],
                                                 sem_d))
            return cps

        def _compute(b):
            @pl.loop(0, CH // 16)
            def _group(g):
                sl = pl.ds(g * 16, 16)
                isrc = src_v[b, sl]
                idst = dst_v[b, sl]
                w0 = _leaky_exp(plsc.load_gather(as0_v, [isrc]) +
                                plsc.load_gather(ad0_v, [idst]))
                w0_v[b, sl] = w0
                if two_heads:
                    w1 = _leaky_exp(plsc.load_gather(as1_v, [isrc]) +
                                    plsc.load_gather(ad1_v, [idst]))
                    w1_v[b, sl] = w1
                else:
                    w1 = w0
                for j in range(16):
                    ei = g * 16 + j
                    bj = jnp.full((16,), j, jnp.int32)
                    b0 = jnp.take_along_axis(w0, bj, axis=0)
                    b1 = jnp.take_along_axis(w1, bj, axis=0) if two_heads \
                        else b0
                    rows_v[b, ei, pl.ds(0, 16)] = (
                        rows_v[b, ei, pl.ds(0, 16)] * b0)
                    rows_v[b, ei, pl.ds(nh, 16)] = (
                        rows_v[b, ei, pl.ds(nh, 16)] * b1)

        @pl.when(_valid(0))
        def _prologue():
            ca, cb = _idx_copies(0, 0)
            ca.start()
            cb.start()
            ca.wait()
            cb.wait()
            _gather_copy(0).start()

        @pl.when(_valid(1))
        def _prologue2():
            ca, cb = _idx_copies(1, 1)
            ca.start()
            cb.start()

        tpipe = (cpt + 2 + NBUF - 1) & ~(NBUF - 1)

        @pl.loop(0, tpipe, step=NBUF)
        def _pipe(tt):
            for b in range(NBUF):
                t = tt + b
                b1 = (b + 1) % NBUF
                b2 = (b + 2) % NBUF

                @pl.when((t >= 2) & _valid(t - 2))
                def _drain():
                    for cp in _scatter_copies(b2):
                        cp.wait()

                @pl.when(_valid(t + 2))
                def _pref_idx():
                    ca, cb = _idx_copies(t + 2, b2)
                    ca.start()
                    cb.start()

                @pl.when(_valid(t + 1))
                def _pref_rows():
                    ca, cb = _idx_copies(t + 1, b1)
                    ca.wait()
                    cb.wait()
                    _gather_copy(b1).start()

                @pl.when(_valid(t))
                def _work():
                    _gather_copy(b).wait()
                    _compute(b)
                    for cp in _scatter_copies(b):
                        cp.start(add=True)
        plsc.subcore_barrier()

        def wout(k, _):
            ch = s + k * NS
            @pl.when(ch < zch)
            def _do():
                sl = pl.ds(ch * CH, CH)
                flat = pl.ds(c * npad + ch * CH, CH)
                pltpu.async_copy(numer_sh.at[sl], numer_o.at[c].at[sl], sem_d)
                pltpu.async_copy(den0_sh.at[sl], den0_o.at[flat], sem_d)
                if two_heads:
                    pltpu.async_copy(den1_sh.at[sl], den1_o.at[flat], sem_d)
            return _
        lax.fori_loop(0, zpt, wout, None)

        def wwait(k, _):
            ch = s + k * NS
            @pl.when(ch < zch)
            def _do():
                sl = pl.ds(ch * CH, CH)
                flat = pl.ds(c * npad + ch * CH, CH)
                pltpu.make_async_copy(numer_sh.at[sl], numer_o.at[c].at[sl],
                                      sem_d).wait()
                pltpu.make_async_copy(den0_sh.at[sl], den0_o.at[flat],
                                      sem_d).wait()
                if two_heads:
                    pltpu.make_async_copy(den1_sh.at[sl], den1_o.at[flat],
                                          sem_d).wait()
            return _
        lax.fori_loop(0, zpt, wwait, None)

    return pl.kernel(
        body, out_type=out_type, mesh=mesh, scratch_types=scratch,
        compiler_params=pltpu.CompilerParams(needs_layout_passes=False,
                                             use_tc_tiling_on_sc=False))


@jax.jit
def kernel(h, edge_index, W0, a0, W1, a1, W_out, a_out):
    n, _ = h.shape
    e = edge_index.shape[1]
    src = edge_index[0]
    dst = edge_index[1]
    f2 = 2 * W0.shape[1]
    nc2 = W_out.shape[1]
    npad = ((n + CH - 1) // CH) * CH

    wh01, as0, ad0, as1, ad1 = _tc_layer0(h, W0, W1, a0, a1, npad)
    as0, ad0, as1, ad1 = (v.reshape(-1) for v in (as0, ad0, as1, ad1))
    numer, den0, den1 = _sc_edge(True, npad, f2, e)(
        wh01, as0, ad0, as1, ad1, src, dst)
    whx, aso, ado = _tc_mid(numer, den0.reshape(NC, npad),
                            den1.reshape(NC, npad), W_out, a_out)
    aso, ado = aso.reshape(-1), ado.reshape(-1)
    numer_o, den_o, _unused = _sc_edge(False, npad, nc2, e)(
        whx, aso, ado, aso, ado, src, dst)
    return _tc_final(numer_o, den_o.reshape(NC, npad), n)


# speedup vs baseline: 1.0092x; 1.0092x over previous
import functools

import jax
import jax.numpy as jnp
from jax import lax
from jax.experimental import pallas as pl
from jax.experimental.pallas import tpu as pltpu
from jax.experimental.pallas import tpu_sc as plsc

ALPHA = 0.2
NC, NS = 2, 16
NW = NC * NS
CH = 256
NBUF = 6


def _leaky_exp(x):
    return jnp.exp(jnp.maximum(x, ALPHA * x))


def _tc_layer0(h, W0, W1, a0, a1, npad, grid=8):
    n, d = h.shape
    nh = W0.shape[1]
    f2 = 2 * nh
    blk = npad // grid

    def body(h_ref, w0_ref, w1_ref, a0_ref, a1_ref, wh_ref, as0_ref,
             ad0_ref, as1_ref, ad1_ref):
        wh0 = h_ref[...] @ w0_ref[...]
        wh1 = h_ref[...] @ w1_ref[...]
        wh_ref[...] = jnp.concatenate([wh0, wh1], axis=1)
        as0_ref[...] = wh0 @ a0_ref[...][:nh, :]
        ad0_ref[...] = wh0 @ a0_ref[...][nh:, :]
        as1_ref[...] = wh1 @ a1_ref[...][:nh, :]
        ad1_ref[...] = wh1 @ a1_ref[...][nh:, :]

    vec = jax.ShapeDtypeStruct((npad, 1), jnp.float32)
    return pl.pallas_call(
        body,
        grid=(grid,),
        in_specs=[
            pl.BlockSpec((blk, d), lambda i: (i, 0)),
            pl.BlockSpec((d, nh), lambda i: (0, 0)),
            pl.BlockSpec((d, nh), lambda i: (0, 0)),
            pl.BlockSpec(a0.shape, lambda i: (0, 0)),
            pl.BlockSpec(a1.shape, lambda i: (0, 0)),
        ],
        out_specs=[
            pl.BlockSpec((blk, f2), lambda i: (i, 0)),
            pl.BlockSpec((blk, 1), lambda i: (i, 0)),
            pl.BlockSpec((blk, 1), lambda i: (i, 0)),
            pl.BlockSpec((blk, 1), lambda i: (i, 0)),
            pl.BlockSpec((blk, 1), lambda i: (i, 0)),
        ],
        out_shape=[jax.ShapeDtypeStruct((npad, f2), jnp.float32),
                   vec, vec, vec, vec],
    )(h, W0, W1, a0, a1)


def _tc_mid(numer, den0, den1, W_out, a_out, grid=8):
    _, npad, f2 = numer.shape
    nh = f2 // 2
    nc = W_out.shape[1]
    blk = npad // grid

    def body(num_ref, d0_ref, d1_ref, w_ref, a_ref, whx_ref, aso_ref,
             ado_ref):
        num = num_ref[0] + num_ref[1]
        d0 = d0_ref[0, :] + d0_ref[1, :]
        d1 = d1_ref[0, :] + d1_ref[1, :]
        d0 = jnp.where(d0 == 0.0, 1.0, d0)
        d1 = jnp.where(d1 == 0.0, 1.0, d1)
        x0 = num[:, :nh] / d0[:, None]
        x1 = num[:, nh:] / d1[:, None]
        x = jnp.concatenate([x0, x1], axis=1)
        x = jnp.where(x > 0.0, x, jnp.exp(x) - 1.0)
        whx = x @ w_ref[...]
        whx_ref[...] = whx
        aso_ref[...] = whx @ a_ref[...][:nc, :]
        ado_ref[...] = whx @ a_ref[...][nc:, :]

    vec = jax.ShapeDtypeStruct((npad, 1), jnp.float32)
    return pl.pallas_call(
        body,
        grid=(grid,),
        in_specs=[
            pl.BlockSpec((2, blk, f2), lambda i: (0, i, 0)),
            pl.BlockSpec((2, blk), lambda i: (0, i)),
            pl.BlockSpec((2, blk), lambda i: (0, i)),
            pl.BlockSpec(W_out.shape, lambda i: (0, 0)),
            pl.BlockSpec(a_out.shape, lambda i: (0, 0)),
        ],
        out_specs=[
            pl.BlockSpec((blk, nc), lambda i: (i, 0)),
            pl.BlockSpec((blk, 1), lambda i: (i, 0)),
            pl.BlockSpec((blk, 1), lambda i: (i, 0)),
        ],
        out_shape=[jax.ShapeDtypeStruct((npad, nc), jnp.float32), vec, vec],
    )(numer, den0, den1, W_out, a_out)


def _tc_final(numer, den, n, grid=8):
    _, npad, nc = numer.shape
    blk = npad // grid

    def body(num_ref, d_ref, out_ref):
        num = num_ref[0] + num_ref[1]
        d = d_ref[0, :] + d_ref[1, :]
        d = jnp.where(d == 0.0, 1.0, d)
        x = num / d[:, None]
        x = x - jnp.max(x, axis=1, keepdims=True)
        out_ref[...] = x - jnp.log(jnp.sum(jnp.exp(x), axis=1, keepdims=True))

    return pl.pallas_call(
        body,
        grid=(grid,),
        in_specs=[
            pl.BlockSpec((2, blk, nc), lambda i: (0, i, 0)),
            pl.BlockSpec((2, blk), lambda i: (0, i)),
        ],
        out_specs=pl.BlockSpec((blk, nc), lambda i: (i, 0)),
        out_shape=jax.ShapeDtypeStruct((n, nc), jnp.float32),
    )(numer, den)


def _sc_edge(two_heads, npad, f2, e):
    nh = f2 // 2
    nchunks = e // CH
    cpt = (nchunks + NW - 1) // NW
    zch = npad // CH
    zpt = (zch + NS - 1) // NS

    mesh = plsc.VectorSubcoreMesh(core_axis_name="c", subcore_axis_name="s",
                                  num_cores=NC)
    vecf = jax.ShapeDtypeStruct((NC * npad,), jnp.float32)
    out_type = [jax.ShapeDtypeStruct((NC, npad, f2), jnp.float32), vecf, vecf]
    scratch = [
        pltpu.VMEM((npad,), jnp.float32),
        pltpu.VMEM((npad,), jnp.float32),
        pltpu.VMEM((npad,), jnp.float32),
        pltpu.VMEM((npad,), jnp.float32),
        pltpu.VMEM((NBUF, CH), jnp.int32),
        pltpu.VMEM((NBUF, CH), jnp.int32),
        pltpu.VMEM((NBUF, CH, f2), jnp.float32),
        pltpu.VMEM((NBUF, CH), jnp.float32),
        pltpu.VMEM((NBUF, CH), jnp.float32),
        pltpu.VMEM_SHARED((npad, f2), jnp.float32),
        pltpu.VMEM_SHARED((npad,), jnp.float32),
        pltpu.VMEM_SHARED((npad,), jnp.float32),
        pltpu.SemaphoreType.DMA,
        pltpu.SemaphoreType.DMA,
        pltpu.SemaphoreType.DMA,
    ]

    def body(wh_h, as0_h, ad0_h, as1_h, ad1_h, src_h, dst_h,
             numer_o, den0_o, den1_o,
             as0_v, ad0_v, as1_v, ad1_v, src_v, dst_v, rows_v, w0_v, w1_v,
             numer_sh, den0_sh, den1_sh, sem_a, sem_b, sem_d):
        c = lax.axis_index("c")
        s = lax.axis_index("s")
        zero16 = jnp.zeros((16,), jnp.float32)

        alpha_cps = [pltpu.make_async_copy(as0_h, as0_v, sem_a),
                     pltpu.make_async_copy(ad0_h, ad0_v, sem_a)]
        if two_heads:
            alpha_cps += [pltpu.make_async_copy(as1_h, as1_v, sem_a),
                          pltpu.make_async_copy(ad1_h, ad1_v, sem_a)]
        for cp in alpha_cps:
            cp.start()

        def zrow(i, _):
            rows_v[0, i, pl.ds(0, 16)] = zero16
            rows_v[0, i, pl.ds(nh, 16)] = zero16
            return _
        lax.fori_loop(0, CH, zrow, None)
        for g in range(CH // 16):
            w0_v[0, pl.ds(g * 16, 16)] = zero16

        def zacc(k, _):
            ch = s + k * NS
            @pl.when(ch < zch)
            def _do():
                sl = pl.ds(ch * CH, CH)
                pltpu.async_copy(rows_v.at[0], numer_sh.at[sl], sem_d)
                pltpu.async_copy(w0_v.at[0], den0_sh.at[sl], sem_d)
                if two_heads:
                    pltpu.async_copy(w0_v.at[0], den1_sh.at[sl], sem_d)
            return _
        lax.fori_loop(0, zpt, zacc, None)

        def zwait(k, _):
            ch = s + k * NS
            @pl.when(ch < zch)
            def _do():
                sl = pl.ds(ch * CH, CH)
                pltpu.make_async_copy(rows_v.at[0], numer_sh.at[sl],
                                      sem_d).wait()
                pltpu.make_async_copy(w0_v.at[0], den0_sh.at[sl],
                                      sem_d).wait()
                if two_heads:
                    pltpu.make_async_copy(w0_v.at[0], den1_sh.at[sl],
                                          sem_d).wait()
            return _
        lax.fori_loop(0, zpt, zwait, None)
        for cp in alpha_cps:
            cp.wait()
        plsc.subcore_barrier()

        wid = c * NS + s

        def _valid(t):
            return wid + t * NW < nchunks

        def _idx_copies(t, b):
            base = (wid + t * NW) * CH
            return (pltpu.make_async_copy(src_h.at[pl.ds(base, CH)],
                                          src_v.at[b], sem_a),
                    pltpu.make_async_copy(dst_h.at[pl.ds(base, CH)],
                                          dst_v.at[b], sem_a))

        def _gather_copy(b):
            return pltpu.make_async_copy(wh_h.at[src_v.at[b]], rows_v.at[b],
                                         sem_b)

        def _scatter_copies(b):
            cps = [pltpu.make_async_copy(rows_v.at[b],
                                         numer_sh.at[dst_v.at[b]], sem_d),
                   pltpu.make_async_copy(w0_v.at[b],
                                         den0_sh.at[dst_v.at[b]], sem_d)]
            if two_heads:
                cps.append(pltpu.make_async_copy(w1_v.at[b],
                                                 den1_sh.at[dst_v.at[b]],
                                                 sem_d))
            return cps

        def _compute(b):
            @pl.loop(0, CH // 16)
            def _group(g):
                sl = pl.ds(g * 16, 16)
                isrc = src_v[b, sl]
                idst = dst_v[b, sl]
                w0 = _leaky_exp(plsc.load_gather(as0_v, [isrc]) +
                                plsc.load_gather(ad0_v, [idst]))
                w0_v[b, sl] = w0
                if two_heads:
                    w1 = _leaky_exp(plsc.load_gather(as1_v, [isrc]) +
                                    plsc.load_gather(ad1_v, [idst]))
                    w1_v[b, sl] = w1
                else:
                    w1 = w0
                for j in range(16):
                    ei = g * 16 + j
                    bj = jnp.full((16,), j, jnp.int32)
                    b0 = jnp.take_along_axis(w0, bj, axis=0)
                    b1 = jnp.take_along_axis(w1, bj, axis=0) if two_heads \
                        else b0
                    rows_v[b, ei, pl.ds(0, 16)] = (
                        rows_v[b, ei, pl.ds(0, 16)] * b0)
                    rows_v[b, ei, pl.ds(nh, 16)] = (
                        rows_v[b, ei, pl.ds(nh, 16)] * b1)

        @pl.when(_valid(0))
        def _prologue():
            ca, cb = _idx_copies(0, 0)
            ca.start()
            cb.start()
            ca.wait()
            cb.wait()
            _gather_copy(0).start()

        @pl.when(_valid(1))
        def _prologue2():
            ca, cb = _idx_copies(1, 1)
            ca.start()
            cb.start()

        tpipe = ((cpt + 2 + NBUF - 1) // NBUF) * NBUF

        @pl.loop(0, tpipe, step=NBUF)
        def _pipe(tt):
            for b in range(NBUF):
                t = tt + b
                b1 = (b + 1) % NBUF
                b2 = (b + 2) % NBUF

                @pl.when((t >= 2) & _valid(t - 2))
                def _drain():
                    for cp in _scatter_copies(b2):
                        cp.wait()

                @pl.when(_valid(t + 2))
                def _pref_idx():
                    ca, cb = _idx_copies(t + 2, b2)
                    ca.start()
                    cb.start()

                @pl.when(_valid(t + 1))
                def _pref_rows():
                    ca, cb = _idx_copies(t + 1, b1)
                    ca.wait()
                    cb.wait()
                    _gather_copy(b1).start()

                @pl.when(_valid(t))
                def _work():
                    _gather_copy(b).wait()
                    _compute(b)
                    for cp in _scatter_copies(b):
                        cp.start(add=True)
        plsc.subcore_barrier()

        def wout(k, _):
            ch = s + k * NS
            @pl.when(ch < zch)
            def _do():
                sl = pl.ds(ch * CH, CH)
                flat = pl.ds(c * npad + ch * CH, CH)
                pltpu.async_copy(numer_sh.at[sl], numer_o.at[c].at[sl], sem_d)
                pltpu.async_copy(den0_sh.at[sl], den0_o.at[flat], sem_d)
                if two_heads:
                    pltpu.async_copy(den1_sh.at[sl], den1_o.at[flat], sem_d)
            return _
        lax.fori_loop(0, zpt, wout, None)

        def wwait(k, _):
            ch = s + k * NS
            @pl.when(ch < zch)
            def _do():
                sl = pl.ds(ch * CH, CH)
                flat = pl.ds(c * npad + ch * CH, CH)
                pltpu.make_async_copy(numer_sh.at[sl], numer_o.at[c].at[sl],
                                      sem_d).wait()
                pltpu.make_async_copy(den0_sh.at[sl], den0_o.at[flat],
                                      sem_d).wait()
                if two_heads:
                    pltpu.make_async_copy(den1_sh.at[sl], den1_o.at[flat],
                                          sem_d).wait()
            return _
        lax.fori_loop(0, zpt, wwait, None)

    return pl.kernel(
        body, out_type=out_type, mesh=mesh, scratch_types=scratch,
        compiler_params=pltpu.CompilerParams(needs_layout_passes=False,
                                             use_tc_tiling_on_sc=False))


@jax.jit
def kernel(h, edge_index, W0, a0, W1, a1, W_out, a_out):
    n, _ = h.shape
    e = edge_index.shape[1]
    src = edge_index[0]
    dst = edge_index[1]
    f2 = 2 * W0.shape[1]
    nc2 = W_out.shape[1]
    npad = ((n + CH - 1) // CH) * CH

    wh01, as0, ad0, as1, ad1 = _tc_layer0(h, W0, W1, a0, a1, npad)
    as0, ad0, as1, ad1 = (v.reshape(-1) for v in (as0, ad0, as1, ad1))
    numer, den0, den1 = _sc_edge(True, npad, f2, e)(
        wh01, as0, ad0, as1, ad1, src, dst)
    whx, aso, ado = _tc_mid(numer, den0.reshape(NC, npad),
                            den1.reshape(NC, npad), W_out, a_out)
    aso, ado = aso.reshape(-1), ado.reshape(-1)
    numer_o, den_o, _unused = _sc_edge(False, npad, nc2, e)(
        whx, aso, ado, aso, ado, src, dst)
    return _tc_final(numer_o, den_o.reshape(NC, npad), n)


# speedup vs baseline: 1.0361x; 1.0267x over previous
import functools

import jax
import jax.numpy as jnp
from jax import lax
from jax.experimental import pallas as pl
from jax.experimental.pallas import tpu as pltpu
from jax.experimental.pallas import tpu_sc as plsc

ALPHA = 0.2
NC, NS = 2, 16
NW = NC * NS
CH = 256
NBUF = 4


def _leaky_exp(x):
    return jnp.exp(jnp.maximum(x, ALPHA * x))


def _tc_layer0(h, W0, W1, a0, a1, npad, grid=4):
    n, d = h.shape
    nh = W0.shape[1]
    f2 = 2 * nh
    blk = npad // grid

    def body(h_ref, w0_ref, w1_ref, a0_ref, a1_ref, wh_ref, as0_ref,
             ad0_ref, as1_ref, ad1_ref):
        wh0 = h_ref[...] @ w0_ref[...]
        wh1 = h_ref[...] @ w1_ref[...]
        wh_ref[...] = jnp.concatenate([wh0, wh1], axis=1)
        as0_ref[...] = wh0 @ a0_ref[...][:nh, :]
        ad0_ref[...] = wh0 @ a0_ref[...][nh:, :]
        as1_ref[...] = wh1 @ a1_ref[...][:nh, :]
        ad1_ref[...] = wh1 @ a1_ref[...][nh:, :]

    vec = jax.ShapeDtypeStruct((npad, 1), jnp.float32)
    return pl.pallas_call(
        body,
        grid=(grid,),
        in_specs=[
            pl.BlockSpec((blk, d), lambda i: (i, 0)),
            pl.BlockSpec((d, nh), lambda i: (0, 0)),
            pl.BlockSpec((d, nh), lambda i: (0, 0)),
            pl.BlockSpec(a0.shape, lambda i: (0, 0)),
            pl.BlockSpec(a1.shape, lambda i: (0, 0)),
        ],
        out_specs=[
            pl.BlockSpec((blk, f2), lambda i: (i, 0)),
            pl.BlockSpec((blk, 1), lambda i: (i, 0)),
            pl.BlockSpec((blk, 1), lambda i: (i, 0)),
            pl.BlockSpec((blk, 1), lambda i: (i, 0)),
            pl.BlockSpec((blk, 1), lambda i: (i, 0)),
        ],
        out_shape=[jax.ShapeDtypeStruct((npad, f2), jnp.float32),
                   vec, vec, vec, vec],
    )(h, W0, W1, a0, a1)


def _tc_mid(numer, den0, den1, W_out, a_out, grid=4):
    _, npad, f2 = numer.shape
    nh = f2 // 2
    nc = W_out.shape[1]
    blk = npad // grid

    def body(num_ref, d0_ref, d1_ref, w_ref, a_ref, whx_ref, aso_ref,
             ado_ref):
        num = num_ref[0] + num_ref[1]
        d0 = d0_ref[0, :] + d0_ref[1, :]
        d1 = d1_ref[0, :] + d1_ref[1, :]
        d0 = jnp.where(d0 == 0.0, 1.0, d0)
        d1 = jnp.where(d1 == 0.0, 1.0, d1)
        x0 = num[:, :nh] / d0[:, None]
        x1 = num[:, nh:] / d1[:, None]
        x = jnp.concatenate([x0, x1], axis=1)
        x = jnp.where(x > 0.0, x, jnp.exp(x) - 1.0)
        whx = x @ w_ref[...]
        whx_ref[...] = whx
        aso_ref[...] = whx @ a_ref[...][:nc, :]
        ado_ref[...] = whx @ a_ref[...][nc:, :]

    vec = jax.ShapeDtypeStruct((npad, 1), jnp.float32)
    return pl.pallas_call(
        body,
        grid=(grid,),
        in_specs=[
            pl.BlockSpec((2, blk, f2), lambda i: (0, i, 0)),
            pl.BlockSpec((2, blk), lambda i: (0, i)),
            pl.BlockSpec((2, blk), lambda i: (0, i)),
            pl.BlockSpec(W_out.shape, lambda i: (0, 0)),
            pl.BlockSpec(a_out.shape, lambda i: (0, 0)),
        ],
        out_specs=[
            pl.BlockSpec((blk, nc), lambda i: (i, 0)),
            pl.BlockSpec((blk, 1), lambda i: (i, 0)),
            pl.BlockSpec((blk, 1), lambda i: (i, 0)),
        ],
        out_shape=[jax.ShapeDtypeStruct((npad, nc), jnp.float32), vec, vec],
    )(numer, den0, den1, W_out, a_out)


def _tc_final(numer, den, n, grid=4):
    _, npad, nc = numer.shape
    blk = npad // grid

    def body(num_ref, d_ref, out_ref):
        num = num_ref[0] + num_ref[1]
        d = d_ref[0, :] + d_ref[1, :]
        d = jnp.where(d == 0.0, 1.0, d)
        x = num / d[:, None]
        x = x - jnp.max(x, axis=1, keepdims=True)
        out_ref[...] = x - jnp.log(jnp.sum(jnp.exp(x), axis=1, keepdims=True))

    return pl.pallas_call(
        body,
        grid=(grid,),
        in_specs=[
            pl.BlockSpec((2, blk, nc), lambda i: (0, i, 0)),
            pl.BlockSpec((2, blk), lambda i: (0, i)),
        ],
        out_specs=pl.BlockSpec((blk, nc), lambda i: (i, 0)),
        out_shape=jax.ShapeDtypeStruct((n, nc), jnp.float32),
    )(numer, den)


def _sc_edge(two_heads, npad, f2, e):
    nh = f2 // 2
    nchunks = e // CH
    cpt = (nchunks + NW - 1) // NW
    zch = npad // CH
    zpt = (zch + NS - 1) // NS

    mesh = plsc.VectorSubcoreMesh(core_axis_name="c", subcore_axis_name="s",
                                  num_cores=NC)
    vecf = jax.ShapeDtypeStruct((NC * npad,), jnp.float32)
    out_type = [jax.ShapeDtypeStruct((NC, npad, f2), jnp.float32), vecf, vecf]
    scratch = [
        pltpu.VMEM((npad,), jnp.float32),
        pltpu.VMEM((npad,), jnp.float32),
        pltpu.VMEM((npad,), jnp.float32),
        pltpu.VMEM((npad,), jnp.float32),
        pltpu.VMEM((NBUF, CH), jnp.int32),
        pltpu.VMEM((NBUF, CH), jnp.int32),
        pltpu.VMEM((NBUF, CH, f2), jnp.float32),
        pltpu.VMEM((NBUF, CH), jnp.float32),
        pltpu.VMEM((NBUF, CH), jnp.float32),
        pltpu.VMEM_SHARED((npad, f2), jnp.float32),
        pltpu.VMEM_SHARED((npad,), jnp.float32),
        pltpu.VMEM_SHARED((npad,), jnp.float32),
        pltpu.SemaphoreType.DMA,
        pltpu.SemaphoreType.DMA,
        pltpu.SemaphoreType.DMA,
    ]

    def body(wh_h, as0_h, ad0_h, as1_h, ad1_h, src_h, dst_h,
             numer_o, den0_o, den1_o,
             as0_v, ad0_v, as1_v, ad1_v, src_v, dst_v, rows_v, w0_v, w1_v,
             numer_sh, den0_sh, den1_sh, sem_a, sem_b, sem_d):
        c = lax.axis_index("c")
        s = lax.axis_index("s")
        zero16 = jnp.zeros((16,), jnp.float32)

        alpha_cps = [pltpu.make_async_copy(as0_h, as0_v, sem_a),
                     pltpu.make_async_copy(ad0_h, ad0_v, sem_a)]
        if two_heads:
            alpha_cps += [pltpu.make_async_copy(as1_h, as1_v, sem_a),
                          pltpu.make_async_copy(ad1_h, ad1_v, sem_a)]
        for cp in alpha_cps:
            cp.start()

        def zrow(i, _):
            rows_v[0, i, pl.ds(0, 16)] = zero16
            rows_v[0, i, pl.ds(nh, 16)] = zero16
            return _
        lax.fori_loop(0, CH, zrow, None)
        for g in range(CH // 16):
            w0_v[0, pl.ds(g * 16, 16)] = zero16

        def zacc(k, _):
            ch = s + k * NS
            @pl.when(ch < zch)
            def _do():
                sl = pl.ds(ch * CH, CH)
                pltpu.async_copy(rows_v.at[0], numer_sh.at[sl], sem_d)
                pltpu.async_copy(w0_v.at[0], den0_sh.at[sl], sem_d)
                if two_heads:
                    pltpu.async_copy(w0_v.at[0], den1_sh.at[sl], sem_d)
            return _
        lax.fori_loop(0, zpt, zacc, None)

        def zwait(k, _):
            ch = s + k * NS
            @pl.when(ch < zch)
            def _do():
                sl = pl.ds(ch * CH, CH)
                pltpu.make_async_copy(rows_v.at[0], numer_sh.at[sl],
                                      sem_d).wait()
                pltpu.make_async_copy(w0_v.at[0], den0_sh.at[sl],
                                      sem_d).wait()
                if two_heads:
                    pltpu.make_async_copy(w0_v.at[0], den1_sh.at[sl],
                                          sem_d).wait()
            return _
        lax.fori_loop(0, zpt, zwait, None)
        for cp in alpha_cps:
            cp.wait()
        plsc.subcore_barrier()

        wid = c * NS + s

        def _valid(t):
            return wid + t * NW < nchunks

        def _idx_copies(t, b):
            base = (wid + t * NW) * CH
            return (pltpu.make_async_copy(src_h.at[pl.ds(base, CH)],
                                          src_v.at[b], sem_a),
                    pltpu.make_async_copy(dst_h.at[pl.ds(base, CH)],
                                          dst_v.at[b], sem_a))

        def _gather_copy(b):
            return pltpu.make_async_copy(wh_h.at[src_v.at[b]], rows_v.at[b],
                                         sem_b)

        def _scatter_copies(b):
            cps = [pltpu.make_async_copy(rows_v.at[b],
                                         numer_sh.at[dst_v.at[b]], sem_d),
                   pltpu.make_async_copy(w0_v.at[b],
                                         den0_sh.at[dst_v.at[b]], sem_d)]
            if two_heads:
                cps.append(pltpu.make_async_copy(w1_v.at[b],
                                                 den1_sh.at[dst_v.at[b]],
                                                 sem_d))
            return cps

        def _compute(b):
            @pl.loop(0, CH // 16)
            def _group(g):
                sl = pl.ds(g * 16, 16)
                isrc = src_v[b, sl]
                idst = dst_v[b, sl]
                w0 = _leaky_exp(plsc.load_gather(as0_v, [isrc]) +
                                plsc.load_gather(ad0_v, [idst]))
                w0_v[b, sl] = w0
                if two_heads:
                    w1 = _leaky_exp(plsc.load_gather(as1_v, [isrc]) +
                                    plsc.load_gather(ad1_v, [idst]))
                    w1_v[b, sl] = w1
                else:
                    w1 = w0
                for j in range(16):
                    ei = g * 16 + j
                    bj = jnp.full((16,), j, jnp.int32)
                    b0 = jnp.take_along_axis(w0, bj, axis=0)
                    b1 = jnp.take_along_axis(w1, bj, axis=0) if two_heads \
                        else b0
                    rows_v[b, ei, pl.ds(0, 16)] = (
                        rows_v[b, ei, pl.ds(0, 16)] * b0)
                    rows_v[b, ei, pl.ds(nh, 16)] = (
                        rows_v[b, ei, pl.ds(nh, 16)] * b1)

        @pl.when(_valid(0))
        def _prologue():
            ca, cb = _idx_copies(0, 0)
            ca.start()
            cb.start()
            ca.wait()
            cb.wait()
            _gather_copy(0).start()

        @pl.when(_valid(1))
        def _prologue2():
            ca, cb = _idx_copies(1, 1)
            ca.start()
            cb.start()

        tpipe = ((cpt + 2 + NBUF - 1) // NBUF) * NBUF

        @pl.loop(0, tpipe, step=NBUF)
        def _pipe(tt):
            for b in range(NBUF):
                t = tt + b
                b1 = (b + 1) % NBUF
                b2 = (b + 2) % NBUF

                @pl.when((t >= 2) & _valid(t - 2))
                def _drain():
                    for cp in _scatter_copies(b2):
                        cp.wait()

                @pl.when(_valid(t + 2))
                def _pref_idx():
                    ca, cb = _idx_copies(t + 2, b2)
                    ca.start()
                    cb.start()

                @pl.when(_valid(t + 1))
                def _pref_rows():
                    ca, cb = _idx_copies(t + 1, b1)
                    ca.wait()
                    cb.wait()
                    _gather_copy(b1).start()

                @pl.when(_valid(t))
                def _work():
                    _gather_copy(b).wait()
                    _compute(b)
                    for cp in _scatter_copies(b):
                        cp.start(add=True)
        plsc.subcore_barrier()

        def wout(k, _):
            ch = s + k * NS
            @pl.when(ch < zch)
            def _do():
                sl = pl.ds(ch * CH, CH)
                flat = pl.ds(c * npad + ch * CH, CH)
                pltpu.async_copy(numer_sh.at[sl], numer_o.at[c].at[sl], sem_d)
                pltpu.async_copy(den0_sh.at[sl], den0_o.at[flat], sem_d)
                if two_heads:
                    pltpu.async_copy(den1_sh.at[sl], den1_o.at[flat], sem_d)
            return _
        lax.fori_loop(0, zpt, wout, None)

        def wwait(k, _):
            ch = s + k * NS
            @pl.when(ch < zch)
            def _do():
                sl = pl.ds(ch * CH, CH)
                flat = pl.ds(c * npad + ch * CH, CH)
                pltpu.make_async_copy(numer_sh.at[sl], numer_o.at[c].at[sl],
                                      sem_d).wait()
                pltpu.make_async_copy(den0_sh.at[sl], den0_o.at[flat],
                                      sem_d).wait()
                if two_heads:
                    pltpu.make_async_copy(den1_sh.at[sl], den1_o.at[flat],
                                          sem_d).wait()
            return _
        lax.fori_loop(0, zpt, wwait, None)

    return pl.kernel(
        body, out_type=out_type, mesh=mesh, scratch_types=scratch,
        compiler_params=pltpu.CompilerParams(needs_layout_passes=False,
                                             use_tc_tiling_on_sc=False))


@jax.jit
def kernel(h, edge_index, W0, a0, W1, a1, W_out, a_out):
    n, _ = h.shape
    e = edge_index.shape[1]
    src = edge_index[0]
    dst = edge_index[1]
    f2 = 2 * W0.shape[1]
    nc2 = W_out.shape[1]
    npad = ((n + CH - 1) // CH) * CH

    wh01, as0, ad0, as1, ad1 = _tc_layer0(h, W0, W1, a0, a1, npad)
    as0, ad0, as1, ad1 = (v.reshape(-1) for v in (as0, ad0, as1, ad1))
    numer, den0, den1 = _sc_edge(True, npad, f2, e)(
        wh01, as0, ad0, as1, ad1, src, dst)
    whx, aso, ado = _tc_mid(numer, den0.reshape(NC, npad),
                            den1.reshape(NC, npad), W_out, a_out)
    aso, ado = aso.reshape(-1), ado.reshape(-1)
    numer_o, den_o, _unused = _sc_edge(False, npad, nc2, e)(
        whx, aso, ado, aso, ado, src, dst)
    return _tc_final(numer_o, den_o.reshape(NC, npad), n)


# speedup vs baseline: 1.0426x; 1.0063x over previous
import functools

import jax
import jax.numpy as jnp
from jax import lax
from jax.experimental import pallas as pl
from jax.experimental.pallas import tpu as pltpu
from jax.experimental.pallas import tpu_sc as plsc

ALPHA = 0.2
NC, NS = 2, 16
NW = NC * NS
CH = 256
NBUF = 4


def _leaky_exp(x):
    return jnp.exp(jnp.maximum(x, ALPHA * x))


def _tc_layer0(h, W0, W1, a0, a1, npad, grid=2):
    n, d = h.shape
    nh = W0.shape[1]
    f2 = 2 * nh
    blk = npad // grid

    def body(h_ref, w0_ref, w1_ref, a0_ref, a1_ref, wh_ref, as0_ref,
             ad0_ref, as1_ref, ad1_ref):
        wh0 = h_ref[...] @ w0_ref[...]
        wh1 = h_ref[...] @ w1_ref[...]
        wh_ref[...] = jnp.concatenate([wh0, wh1], axis=1)
        as0_ref[...] = wh0 @ a0_ref[...][:nh, :]
        ad0_ref[...] = wh0 @ a0_ref[...][nh:, :]
        as1_ref[...] = wh1 @ a1_ref[...][:nh, :]
        ad1_ref[...] = wh1 @ a1_ref[...][nh:, :]

    vec = jax.ShapeDtypeStruct((npad, 1), jnp.float32)
    return pl.pallas_call(
        body,
        grid=(grid,),
        in_specs=[
            pl.BlockSpec((blk, d), lambda i: (i, 0)),
            pl.BlockSpec((d, nh), lambda i: (0, 0)),
            pl.BlockSpec((d, nh), lambda i: (0, 0)),
            pl.BlockSpec(a0.shape, lambda i: (0, 0)),
            pl.BlockSpec(a1.shape, lambda i: (0, 0)),
        ],
        out_specs=[
            pl.BlockSpec((blk, f2), lambda i: (i, 0)),
            pl.BlockSpec((blk, 1), lambda i: (i, 0)),
            pl.BlockSpec((blk, 1), lambda i: (i, 0)),
            pl.BlockSpec((blk, 1), lambda i: (i, 0)),
            pl.BlockSpec((blk, 1), lambda i: (i, 0)),
        ],
        out_shape=[jax.ShapeDtypeStruct((npad, f2), jnp.float32),
                   vec, vec, vec, vec],
    )(h, W0, W1, a0, a1)


def _tc_mid(numer, den0, den1, W_out, a_out, grid=2):
    _, npad, f2 = numer.shape
    nh = f2 // 2
    nc = W_out.shape[1]
    blk = npad // grid

    def body(num_ref, d0_ref, d1_ref, w_ref, a_ref, whx_ref, aso_ref,
             ado_ref):
        num = num_ref[0] + num_ref[1]
        d0 = d0_ref[0, :] + d0_ref[1, :]
        d1 = d1_ref[0, :] + d1_ref[1, :]
        d0 = jnp.where(d0 == 0.0, 1.0, d0)
        d1 = jnp.where(d1 == 0.0, 1.0, d1)
        x0 = num[:, :nh] / d0[:, None]
        x1 = num[:, nh:] / d1[:, None]
        x = jnp.concatenate([x0, x1], axis=1)
        x = jnp.where(x > 0.0, x, jnp.exp(x) - 1.0)
        whx = x @ w_ref[...]
        whx_ref[...] = whx
        aso_ref[...] = whx @ a_ref[...][:nc, :]
        ado_ref[...] = whx @ a_ref[...][nc:, :]

    vec = jax.ShapeDtypeStruct((npad, 1), jnp.float32)
    return pl.pallas_call(
        body,
        grid=(grid,),
        in_specs=[
            pl.BlockSpec((2, blk, f2), lambda i: (0, i, 0)),
            pl.BlockSpec((2, blk), lambda i: (0, i)),
            pl.BlockSpec((2, blk), lambda i: (0, i)),
            pl.BlockSpec(W_out.shape, lambda i: (0, 0)),
            pl.BlockSpec(a_out.shape, lambda i: (0, 0)),
        ],
        out_specs=[
            pl.BlockSpec((blk, nc), lambda i: (i, 0)),
            pl.BlockSpec((blk, 1), lambda i: (i, 0)),
            pl.BlockSpec((blk, 1), lambda i: (i, 0)),
        ],
        out_shape=[jax.ShapeDtypeStruct((npad, nc), jnp.float32), vec, vec],
    )(numer, den0, den1, W_out, a_out)


def _tc_final(numer, den, n, grid=2):
    _, npad, nc = numer.shape
    blk = npad // grid

    def body(num_ref, d_ref, out_ref):
        num = num_ref[0] + num_ref[1]
        d = d_ref[0, :] + d_ref[1, :]
        d = jnp.where(d == 0.0, 1.0, d)
        x = num / d[:, None]
        x = x - jnp.max(x, axis=1, keepdims=True)
        out_ref[...] = x - jnp.log(jnp.sum(jnp.exp(x), axis=1, keepdims=True))

    return pl.pallas_call(
        body,
        grid=(grid,),
        in_specs=[
            pl.BlockSpec((2, blk, nc), lambda i: (0, i, 0)),
            pl.BlockSpec((2, blk), lambda i: (0, i)),
        ],
        out_specs=pl.BlockSpec((blk, nc), lambda i: (i, 0)),
        out_shape=jax.ShapeDtypeStruct((n, nc), jnp.float32),
    )(numer, den)


def _sc_edge(two_heads, npad, f2, e):
    nh = f2 // 2
    nchunks = e // CH
    cpt = (nchunks + NW - 1) // NW
    zch = npad // CH
    zpt = (zch + NS - 1) // NS

    mesh = plsc.VectorSubcoreMesh(core_axis_name="c", subcore_axis_name="s",
                                  num_cores=NC)
    vecf = jax.ShapeDtypeStruct((NC * npad,), jnp.float32)
    out_type = [jax.ShapeDtypeStruct((NC, npad, f2), jnp.float32), vecf, vecf]
    scratch = [
        pltpu.VMEM((npad,), jnp.float32),
        pltpu.VMEM((npad,), jnp.float32),
        pltpu.VMEM((npad,), jnp.float32),
        pltpu.VMEM((npad,), jnp.float32),
        pltpu.VMEM((NBUF, CH), jnp.int32),
        pltpu.VMEM((NBUF, CH), jnp.int32),
        pltpu.VMEM((NBUF, CH, f2), jnp.float32),
        pltpu.VMEM((NBUF, CH), jnp.float32),
        pltpu.VMEM((NBUF, CH), jnp.float32),
        pltpu.VMEM_SHARED((npad, f2), jnp.float32),
        pltpu.VMEM_SHARED((npad,), jnp.float32),
        pltpu.VMEM_SHARED((npad,), jnp.float32),
        pltpu.SemaphoreType.DMA,
        pltpu.SemaphoreType.DMA,
        pltpu.SemaphoreType.DMA,
    ]

    def body(wh_h, as0_h, ad0_h, as1_h, ad1_h, src_h, dst_h,
             numer_o, den0_o, den1_o,
             as0_v, ad0_v, as1_v, ad1_v, src_v, dst_v, rows_v, w0_v, w1_v,
             numer_sh, den0_sh, den1_sh, sem_a, sem_b, sem_d):
        c = lax.axis_index("c")
        s = lax.axis_index("s")
        zero16 = jnp.zeros((16,), jnp.float32)

        alpha_cps = [pltpu.make_async_copy(as0_h, as0_v, sem_a),
                     pltpu.make_async_copy(ad0_h, ad0_v, sem_a)]
        if two_heads:
            alpha_cps += [pltpu.make_async_copy(as1_h, as1_v, sem_a),
                          pltpu.make_async_copy(ad1_h, ad1_v, sem_a)]
        for cp in alpha_cps:
            cp.start()

        def zrow(i, _):
            rows_v[0, i, pl.ds(0, 16)] = zero16
            rows_v[0, i, pl.ds(nh, 16)] = zero16
            return _
        lax.fori_loop(0, CH, zrow, None)
        for g in range(CH // 16):
            w0_v[0, pl.ds(g * 16, 16)] = zero16

        def zacc(k, _):
            ch = s + k * NS
            @pl.when(ch < zch)
            def _do():
                sl = pl.ds(ch * CH, CH)
                pltpu.async_copy(rows_v.at[0], numer_sh.at[sl], sem_d)
                pltpu.async_copy(w0_v.at[0], den0_sh.at[sl], sem_d)
                if two_heads:
                    pltpu.async_copy(w0_v.at[0], den1_sh.at[sl], sem_d)
            return _
        lax.fori_loop(0, zpt, zacc, None)

        def zwait(k, _):
            ch = s + k * NS
            @pl.when(ch < zch)
            def _do():
                sl = pl.ds(ch * CH, CH)
                pltpu.make_async_copy(rows_v.at[0], numer_sh.at[sl],
                                      sem_d).wait()
                pltpu.make_async_copy(w0_v.at[0], den0_sh.at[sl],
                                      sem_d).wait()
                if two_heads:
                    pltpu.make_async_copy(w0_v.at[0], den1_sh.at[sl],
                                          sem_d).wait()
            return _
        lax.fori_loop(0, zpt, zwait, None)
        for cp in alpha_cps:
            cp.wait()
        plsc.subcore_barrier()

        wid = c * NS + s

        def _valid(t):
            return wid + t * NW < nchunks

        def _idx_copies(t, b):
            base = (wid + t * NW) * CH
            return (pltpu.make_async_copy(src_h.at[pl.ds(base, CH)],
                                          src_v.at[b], sem_a),
                    pltpu.make_async_copy(dst_h.at[pl.ds(base, CH)],
                                          dst_v.at[b], sem_a))

        def _gather_copy(b):
            return pltpu.make_async_copy(wh_h.at[src_v.at[b]], rows_v.at[b],
                                         sem_b)

        def _scatter_copies(b):
            cps = [pltpu.make_async_copy(rows_v.at[b],
                                         numer_sh.at[dst_v.at[b]], sem_d),
                   pltpu.make_async_copy(w0_v.at[b],
                                         den0_sh.at[dst_v.at[b]], sem_d)]
            if two_heads:
                cps.append(pltpu.make_async_copy(w1_v.at[b],
                                                 den1_sh.at[dst_v.at[b]],
                                                 sem_d))
            return cps

        def _compute(b):
            @pl.loop(0, CH // 16)
            def _group(g):
                sl = pl.ds(g * 16, 16)
                isrc = src_v[b, sl]
                idst = dst_v[b, sl]
                w0 = _leaky_exp(plsc.load_gather(as0_v, [isrc]) +
                                plsc.load_gather(ad0_v, [idst]))
                w0_v[b, sl] = w0
                if two_heads:
                    w1 = _leaky_exp(plsc.load_gather(as1_v, [isrc]) +
                                    plsc.load_gather(ad1_v, [idst]))
                    w1_v[b, sl] = w1
                else:
                    w1 = w0
                for j in range(16):
                    ei = g * 16 + j
                    bj = jnp.full((16,), j, jnp.int32)
                    b0 = jnp.take_along_axis(w0, bj, axis=0)
                    b1 = jnp.take_along_axis(w1, bj, axis=0) if two_heads \
                        else b0
                    rows_v[b, ei, pl.ds(0, 16)] = (
                        rows_v[b, ei, pl.ds(0, 16)] * b0)
                    rows_v[b, ei, pl.ds(nh, 16)] = (
                        rows_v[b, ei, pl.ds(nh, 16)] * b1)

        @pl.when(_valid(0))
        def _prologue():
            ca, cb = _idx_copies(0, 0)
            ca.start()
            cb.start()
            ca.wait()
            cb.wait()
            _gather_copy(0).start()

        @pl.when(_valid(1))
        def _prologue2():
            ca, cb = _idx_copies(1, 1)
            ca.start()
            cb.start()

        tpipe = ((cpt + 2 + NBUF - 1) // NBUF) * NBUF

        @pl.loop(0, tpipe, step=NBUF)
        def _pipe(tt):
            for b in range(NBUF):
                t = tt + b
                b1 = (b + 1) % NBUF
                b2 = (b + 2) % NBUF

                @pl.when((t >= 2) & _valid(t - 2))
                def _drain():
                    for cp in _scatter_copies(b2):
                        cp.wait()

                @pl.when(_valid(t + 2))
                def _pref_idx():
                    ca, cb = _idx_copies(t + 2, b2)
                    ca.start()
                    cb.start()

                @pl.when(_valid(t + 1))
                def _pref_rows():
                    ca, cb = _idx_copies(t + 1, b1)
                    ca.wait()
                    cb.wait()
                    _gather_copy(b1).start()

                @pl.when(_valid(t))
                def _work():
                    _gather_copy(b).wait()
                    _compute(b)
                    for cp in _scatter_copies(b):
                        cp.start(add=True)
        plsc.subcore_barrier()

        def wout(k, _):
            ch = s + k * NS
            @pl.when(ch < zch)
            def _do():
                sl = pl.ds(ch * CH, CH)
                flat = pl.ds(c * npad + ch * CH, CH)
                pltpu.async_copy(numer_sh.at[sl], numer_o.at[c].at[sl], sem_d)
                pltpu.async_copy(den0_sh.at[sl], den0_o.at[flat], sem_d)
                if two_heads:
                    pltpu.async_copy(den1_sh.at[sl], den1_o.at[flat], sem_d)
            return _
        lax.fori_loop(0, zpt, wout, None)

        def wwait(k, _):
            ch = s + k * NS
            @pl.when(ch < zch)
            def _do():
                sl = pl.ds(ch * CH, CH)
                flat = pl.ds(c * npad + ch * CH, CH)
                pltpu.make_async_copy(numer_sh.at[sl], numer_o.at[c].at[sl],
                                      sem_d).wait()
                pltpu.make_async_copy(den0_sh.at[sl], den0_o.at[flat],
                                      sem_d).wait()
                if two_heads:
                    pltpu.make_async_copy(den1_sh.at[sl], den1_o.at[flat],
                                          sem_d).wait()
            return _
        lax.fori_loop(0, zpt, wwait, None)

    return pl.kernel(
        body, out_type=out_type, mesh=mesh, scratch_types=scratch,
        compiler_params=pltpu.CompilerParams(needs_layout_passes=False,
                                             use_tc_tiling_on_sc=False))


@jax.jit
def kernel(h, edge_index, W0, a0, W1, a1, W_out, a_out):
    n, _ = h.shape
    e = edge_index.shape[1]
    src = edge_index[0]
    dst = edge_index[1]
    f2 = 2 * W0.shape[1]
    nc2 = W_out.shape[1]
    npad = ((n + CH - 1) // CH) * CH

    wh01, as0, ad0, as1, ad1 = _tc_layer0(h, W0, W1, a0, a1, npad)
    as0, ad0, as1, ad1 = (v.reshape(-1) for v in (as0, ad0, as1, ad1))
    numer, den0, den1 = _sc_edge(True, npad, f2, e)(
        wh01, as0, ad0, as1, ad1, src, dst)
    whx, aso, ado = _tc_mid(numer, den0.reshape(NC, npad),
                            den1.reshape(NC, npad), W_out, a_out)
    aso, ado = aso.reshape(-1), ado.reshape(-1)
    numer_o, den_o, _unused = _sc_edge(False, npad, nc2, e)(
        whx, aso, ado, aso, ado, src, dst)
    return _tc_final(numer_o, den_o.reshape(NC, npad), n)
